# baseline (speedup 1.0000x reference)
import jax
import jax.numpy as jnp
from jax.experimental import pallas as pl
from jax.experimental.pallas import tpu as pltpu

_SIZES = (1024, 3072)


def _dma_body(x_ref, o_ref, *scratch):
    B = x_ref.shape[0]
    n = B * 2 * len(_SIZES)
    bufs = scratch[:n]
    in_sems = scratch[n : 2 * n]
    out_sems = scratch[2 * n :]
    srcs, dsts = [], []
    for b in range(B):
        for j in range(2):
            off = 0
            for sz in _SIZES:
                srcs.append(x_ref.at[b, 2 * j, pl.ds(off, sz)])
                dsts.append(o_ref.at[b, j, pl.ds(off, sz)])
                off += sz
    gathers = [
        pltpu.make_async_copy(srcs[k], bufs[k], in_sems[k]) for k in range(n)
    ]
    for g in gathers:
        g.start()
    scatters = []
    for k in range(n):
        gathers[k].wait()
        s = pltpu.make_async_copy(bufs[k], dsts[k], out_sems[k])
        s.start()
        scatters.append(s)
    for s in scatters:
        s.wait()


def kernel(x):
    B, N, S, D = x.shape
    n = B * 2 * len(_SIZES)
    return pl.pallas_call(
        _dma_body,
        in_specs=[pl.BlockSpec(memory_space=pl.ANY)],
        out_specs=pl.BlockSpec(memory_space=pl.ANY),
        out_shape=jax.ShapeDtypeStruct((B, 2, S, D), x.dtype),
        scratch_shapes=(
            [
                pltpu.VMEM((sz, D), x.dtype)
                for _ in range(B * 2)
                for sz in _SIZES
            ]
            + [pltpu.SemaphoreType.DMA for _ in range(2 * n)]
        ),
    )(x)


# speedup vs baseline: 1.0058x; 1.0058x over previous
import jax
import jax.numpy as jnp
from jax.experimental import pallas as pl
from jax.experimental.pallas import tpu as pltpu

_NSPLIT = 2


def _dma_body(x_ref, o_ref, *scratch):
    B = x_ref.shape[0]
    S = x_ref.shape[2]
    c = S // _NSPLIT
    n = B * 2 * _NSPLIT
    bufs = scratch[:n]
    in_sems = scratch[n : 2 * n]
    out_sems = scratch[2 * n :]
    srcs, dsts = [], []
    for b in range(B):
        for j in range(2):
            for i in range(_NSPLIT):
                srcs.append(x_ref.at[b, 2 * j, pl.ds(i * c, c)])
                dsts.append(o_ref.at[b, j, pl.ds(i * c, c)])
    gathers = [
        pltpu.make_async_copy(srcs[k], bufs[k], in_sems[k]) for k in range(n)
    ]
    for g in gathers:
        g.start()
    scatters = []
    for k in range(n):
        gathers[k].wait()
        s = pltpu.make_async_copy(bufs[k], dsts[k], out_sems[k])
        s.start()
        scatters.append(s)
    for s in scatters:
        s.wait()


def kernel(x):
    B, N, S, D = x.shape
    c = S // _NSPLIT
    n = B * 2 * _NSPLIT
    return pl.pallas_call(
        _dma_body,
        in_specs=[pl.BlockSpec(memory_space=pl.ANY)],
        out_specs=pl.BlockSpec(memory_space=pl.ANY),
        out_shape=jax.ShapeDtypeStruct((B, 2, S, D), x.dtype),
        scratch_shapes=(
            [pltpu.VMEM((c, D), x.dtype) for _ in range(n)]
            + [pltpu.SemaphoreType.DMA for _ in range(2 * n)]
        ),
    )(x)
